# baseline (speedup 1.0000x reference)
import functools

import jax
import jax.numpy as jnp
import numpy as np
from jax import lax
from jax.experimental import pallas as pl
from jax.experimental.pallas import tpu as pltpu
from jax.experimental.pallas import tpu_sc as plsc

_B = 32
_C = 3
_H = 512
_W = 512
_NCAMS = 32
_NIDENT = 1000
_BC = _B * _C


_ROW_IDX = np.arange(_BC, dtype=np.int32) // _C
_COL_IDX = np.arange(_BC, dtype=np.int32) % _C


def _sc_gather_body(cam_hbm, id_hbm, wcam_hbm, bcam_hbm, wident_hbm,
                    bident_hbm, rowi_hbm, coli_hbm, w_out, b_out, cam_v, id_v,
                    wcam_v, bcam_v, wident_v, bident_v, rowi_v, coli_v, w_v,
                    b_v):
    wid = lax.axis_index("s") * 2 + lax.axis_index("c")

    @pl.when(wid == 0)
    def _():
        pltpu.sync_copy(cam_hbm, cam_v)
        pltpu.sync_copy(id_hbm, id_v)
        pltpu.sync_copy(wcam_hbm, wcam_v)
        pltpu.sync_copy(bcam_hbm, bcam_v)
        pltpu.sync_copy(wident_hbm, wident_v)
        pltpu.sync_copy(bident_hbm, bident_v)
        pltpu.sync_copy(rowi_hbm, rowi_v)
        pltpu.sync_copy(coli_hbm, coli_v)
        for i in range(_BC // 16):
            row = rowi_v[pl.ds(16 * i, 16)]
            col = coli_v[pl.ds(16 * i, 16)]
            cams = plsc.load_gather(cam_v, [row])
            ids = plsc.load_gather(id_v, [row])
            cflat = cams * _C + col
            iflat = ids * _C + col
            wv = (plsc.load_gather(wcam_v, [cflat]) +
                  plsc.load_gather(wident_v, [iflat]))
            bv = (plsc.load_gather(bcam_v, [cflat]) +
                  plsc.load_gather(bident_v, [iflat]))
            w_v[pl.ds(16 * i, 16)] = wv
            b_v[pl.ds(16 * i, 16)] = bv
        pltpu.sync_copy(w_v, w_out)
        pltpu.sync_copy(b_v, b_out)


_sc_gather = functools.partial(
    pl.kernel,
    mesh=plsc.VectorSubcoreMesh(core_axis_name="c", subcore_axis_name="s"),
    compiler_params=pltpu.CompilerParams(needs_layout_passes=False),
    out_type=(jax.ShapeDtypeStruct((_BC,), jnp.float32),
              jax.ShapeDtypeStruct((_BC,), jnp.float32)),
    scratch_types=[
        pltpu.VMEM((_B,), jnp.int32),
        pltpu.VMEM((_B,), jnp.int32),
        pltpu.VMEM((_NCAMS * _C,), jnp.float32),
        pltpu.VMEM((_NCAMS * _C,), jnp.float32),
        pltpu.VMEM((_NIDENT * _C,), jnp.float32),
        pltpu.VMEM((_NIDENT * _C,), jnp.float32),
        pltpu.VMEM((_BC,), jnp.int32),
        pltpu.VMEM((_BC,), jnp.int32),
        pltpu.VMEM((_BC,), jnp.float32),
        pltpu.VMEM((_BC,), jnp.float32),
    ],
)(_sc_gather_body)


_HW = _H * _W
_ROWS_PER_BLOCK = 8


def _scale_bias_body(w_ref, b_ref, img_ref, out_ref):
    out_ref[...] = img_ref[...] * w_ref[...] + b_ref[...]


def kernel(image, camindex, idindex, wcam, bcam, wident, bident):
    w_flat, b_flat = _sc_gather(camindex.astype(jnp.int32),
                                idindex.astype(jnp.int32),
                                wcam.reshape(-1), bcam.reshape(-1),
                                wident.reshape(-1), bident.reshape(-1),
                                jnp.asarray(_ROW_IDX), jnp.asarray(_COL_IDX))
    img2 = image.reshape(_BC, _HW)
    r = _ROWS_PER_BLOCK
    out = pl.pallas_call(
        _scale_bias_body,
        grid=(_BC // r,),
        in_specs=[
            pl.BlockSpec((r, 1), lambda i: (i, 0)),
            pl.BlockSpec((r, 1), lambda i: (i, 0)),
            pl.BlockSpec((r, _HW), lambda i: (i, 0)),
        ],
        out_specs=pl.BlockSpec((r, _HW), lambda i: (i, 0)),
        out_shape=jax.ShapeDtypeStruct((_BC, _HW), jnp.float32),
        compiler_params=pltpu.CompilerParams(
            dimension_semantics=("parallel",)),
    )(w_flat.reshape(_BC, 1), b_flat.reshape(_BC, 1), img2)
    return out.reshape(_B, _C, _H, _W)


# speedup vs baseline: 3.2143x; 3.2143x over previous
import functools

import jax
import jax.numpy as jnp
import numpy as np
from jax import lax
from jax.experimental import pallas as pl
from jax.experimental.pallas import tpu as pltpu
from jax.experimental.pallas import tpu_sc as plsc

_B = 32
_C = 3
_H = 512
_W = 512
_NCAMS = 32
_NIDENT = 1000
_BC = _B * _C


_ROW_IDX = np.arange(_BC, dtype=np.int32) // _C
_COL_IDX = np.arange(_BC, dtype=np.int32) % _C


def _sc_gather_body(cam_hbm, id_hbm, wcam_hbm, bcam_hbm, wident_hbm,
                    bident_hbm, rowi_hbm, coli_hbm, w_out, b_out, cam_v, id_v,
                    wcam_v, bcam_v, wident_v, bident_v, rowi_v, coli_v, w_v,
                    b_v):
    wid = lax.axis_index("s") * 2 + lax.axis_index("c")

    @pl.when(wid == 0)
    def _():
        pltpu.sync_copy(cam_hbm, cam_v)
        pltpu.sync_copy(id_hbm, id_v)
        pltpu.sync_copy(wcam_hbm, wcam_v)
        pltpu.sync_copy(bcam_hbm, bcam_v)
        pltpu.sync_copy(wident_hbm, wident_v)
        pltpu.sync_copy(bident_hbm, bident_v)
        pltpu.sync_copy(rowi_hbm, rowi_v)
        pltpu.sync_copy(coli_hbm, coli_v)
        for i in range(_BC // 16):
            row = rowi_v[pl.ds(16 * i, 16)]
            col = coli_v[pl.ds(16 * i, 16)]
            cams = plsc.load_gather(cam_v, [row])
            ids = plsc.load_gather(id_v, [row])
            cflat = cams * _C + col
            iflat = ids * _C + col
            wv = (plsc.load_gather(wcam_v, [cflat]) +
                  plsc.load_gather(wident_v, [iflat]))
            bv = (plsc.load_gather(bcam_v, [cflat]) +
                  plsc.load_gather(bident_v, [iflat]))
            w_v[pl.ds(16 * i, 16)] = wv
            b_v[pl.ds(16 * i, 16)] = bv
        pltpu.sync_copy(w_v, w_out)
        pltpu.sync_copy(b_v, b_out)


_sc_gather = functools.partial(
    pl.kernel,
    mesh=plsc.VectorSubcoreMesh(core_axis_name="c", subcore_axis_name="s"),
    compiler_params=pltpu.CompilerParams(needs_layout_passes=False),
    out_type=(jax.ShapeDtypeStruct((_BC,), jnp.float32),
              jax.ShapeDtypeStruct((_BC,), jnp.float32)),
    scratch_types=[
        pltpu.VMEM((_B,), jnp.int32),
        pltpu.VMEM((_B,), jnp.int32),
        pltpu.VMEM((_NCAMS * _C,), jnp.float32),
        pltpu.VMEM((_NCAMS * _C,), jnp.float32),
        pltpu.VMEM((_NIDENT * _C,), jnp.float32),
        pltpu.VMEM((_NIDENT * _C,), jnp.float32),
        pltpu.VMEM((_BC,), jnp.int32),
        pltpu.VMEM((_BC,), jnp.int32),
        pltpu.VMEM((_BC,), jnp.float32),
        pltpu.VMEM((_BC,), jnp.float32),
    ],
)(_sc_gather_body)


_ROWS_PER_BLOCK = 8


def _scale_bias_body(w_ref, b_ref, img_ref, out_ref):
    i = pl.program_id(0)
    r = _ROWS_PER_BLOCK
    for j in range(r):
        out_ref[j] = img_ref[j] * w_ref[r * i + j] + b_ref[r * i + j]


def kernel(image, camindex, idindex, wcam, bcam, wident, bident):
    w_flat, b_flat = _sc_gather(camindex.astype(jnp.int32),
                                idindex.astype(jnp.int32),
                                wcam.reshape(-1), bcam.reshape(-1),
                                wident.reshape(-1), bident.reshape(-1),
                                jnp.asarray(_ROW_IDX), jnp.asarray(_COL_IDX))
    img3 = image.reshape(_BC, _H, _W)
    r = _ROWS_PER_BLOCK
    out = pl.pallas_call(
        _scale_bias_body,
        grid=(_BC // r,),
        in_specs=[
            pl.BlockSpec(memory_space=pltpu.SMEM),
            pl.BlockSpec(memory_space=pltpu.SMEM),
            pl.BlockSpec((r, _H, _W), lambda i: (i, 0, 0)),
        ],
        out_specs=pl.BlockSpec((r, _H, _W), lambda i: (i, 0, 0)),
        out_shape=jax.ShapeDtypeStruct((_BC, _H, _W), jnp.float32),
        compiler_params=pltpu.CompilerParams(
            dimension_semantics=("parallel",)),
    )(w_flat, b_flat, img3)
    return out.reshape(_B, _C, _H, _W)


# speedup vs baseline: 3.2800x; 1.0205x over previous
import functools

import jax
import jax.numpy as jnp
import numpy as np
from jax import lax
from jax.experimental import pallas as pl
from jax.experimental.pallas import tpu as pltpu
from jax.experimental.pallas import tpu_sc as plsc

_B = 32
_C = 3
_H = 512
_W = 512
_NCAMS = 32
_NIDENT = 1000
_BC = _B * _C


_ROW_IDX = np.arange(_BC, dtype=np.int32) // _C
_COL_IDX = np.arange(_BC, dtype=np.int32) % _C


def _sc_gather_body(cam_hbm, id_hbm, wcam_hbm, bcam_hbm, wident_hbm,
                    bident_hbm, rowi_hbm, coli_hbm, w_out, b_out, cam_v, id_v,
                    wcam_v, bcam_v, wident_v, bident_v, rowi_v, coli_v, w_v,
                    b_v, sem):
    wid = lax.axis_index("s") * 2 + lax.axis_index("c")

    @pl.when(wid == 0)
    def _():
        copies = [
            pltpu.async_copy(cam_hbm, cam_v, sem),
            pltpu.async_copy(id_hbm, id_v, sem),
            pltpu.async_copy(wcam_hbm, wcam_v, sem),
            pltpu.async_copy(bcam_hbm, bcam_v, sem),
            pltpu.async_copy(wident_hbm, wident_v, sem),
            pltpu.async_copy(bident_hbm, bident_v, sem),
            pltpu.async_copy(rowi_hbm, rowi_v, sem),
            pltpu.async_copy(coli_hbm, coli_v, sem),
        ]
        for c in copies:
            c.wait()
        for i in range(_BC // 16):
            row = rowi_v[pl.ds(16 * i, 16)]
            col = coli_v[pl.ds(16 * i, 16)]
            cams = plsc.load_gather(cam_v, [row])
            ids = plsc.load_gather(id_v, [row])
            cflat = cams * _C + col
            iflat = ids * _C + col
            wv = (plsc.load_gather(wcam_v, [cflat]) +
                  plsc.load_gather(wident_v, [iflat]))
            bv = (plsc.load_gather(bcam_v, [cflat]) +
                  plsc.load_gather(bident_v, [iflat]))
            w_v[pl.ds(16 * i, 16)] = wv
            b_v[pl.ds(16 * i, 16)] = bv
        outs = [pltpu.async_copy(w_v, w_out, sem),
                pltpu.async_copy(b_v, b_out, sem)]
        for c in outs:
            c.wait()


_sc_gather = functools.partial(
    pl.kernel,
    mesh=plsc.VectorSubcoreMesh(core_axis_name="c", subcore_axis_name="s"),
    compiler_params=pltpu.CompilerParams(needs_layout_passes=False),
    out_type=(jax.ShapeDtypeStruct((_BC,), jnp.float32),
              jax.ShapeDtypeStruct((_BC,), jnp.float32)),
    scratch_types=[
        pltpu.VMEM((_B,), jnp.int32),
        pltpu.VMEM((_B,), jnp.int32),
        pltpu.VMEM((_NCAMS * _C,), jnp.float32),
        pltpu.VMEM((_NCAMS * _C,), jnp.float32),
        pltpu.VMEM((_NIDENT * _C,), jnp.float32),
        pltpu.VMEM((_NIDENT * _C,), jnp.float32),
        pltpu.VMEM((_BC,), jnp.int32),
        pltpu.VMEM((_BC,), jnp.int32),
        pltpu.VMEM((_BC,), jnp.float32),
        pltpu.VMEM((_BC,), jnp.float32),
        pltpu.SemaphoreType.DMA,
    ],
)(_sc_gather_body)


_ROWS_PER_BLOCK = 4


def _scale_bias_body(w_ref, b_ref, img_ref, out_ref):
    i = pl.program_id(0)
    r = _ROWS_PER_BLOCK
    for j in range(r):
        out_ref[j] = img_ref[j] * w_ref[r * i + j] + b_ref[r * i + j]


def kernel(image, camindex, idindex, wcam, bcam, wident, bident):
    w_flat, b_flat = _sc_gather(camindex.astype(jnp.int32),
                                idindex.astype(jnp.int32),
                                wcam.reshape(-1), bcam.reshape(-1),
                                wident.reshape(-1), bident.reshape(-1),
                                jnp.asarray(_ROW_IDX), jnp.asarray(_COL_IDX))
    img3 = image.reshape(_BC, _H, _W)
    r = _ROWS_PER_BLOCK
    out = pl.pallas_call(
        _scale_bias_body,
        grid=(_BC // r,),
        in_specs=[
            pl.BlockSpec(memory_space=pltpu.SMEM),
            pl.BlockSpec(memory_space=pltpu.SMEM),
            pl.BlockSpec((r, _H, _W), lambda i: (i, 0, 0)),
        ],
        out_specs=pl.BlockSpec((r, _H, _W), lambda i: (i, 0, 0)),
        out_shape=jax.ShapeDtypeStruct((_BC, _H, _W), jnp.float32),
        compiler_params=pltpu.CompilerParams(
            dimension_semantics=("parallel",)),
    )(w_flat, b_flat, img3)
    return out.reshape(_B, _C, _H, _W)
